# baseline (speedup 1.0000x reference)
import functools

import jax
import jax.numpy as jnp
from jax import lax
from jax.experimental import pallas as pl
from jax.experimental.pallas import tpu as pltpu
from jax.experimental.pallas import tpu_sc as plsc

_VOCAB = 1000001
_H = 501760
_B = 16384
_L = 50
_D = 300
_C = 2 * _L
_CP = 128
_BM = 2048

_NC = 2
_NS = 16
_NW = _NC * _NS
_BPW = _B // _NW
_G = 128
_NG = _C * _BPW // _G
_K = 16


def _rne16(d):
    u = jax.lax.bitcast_convert_type(d, jnp.uint32)
    r = u + jnp.uint32(0x7FFF) + ((u >> jnp.uint32(16)) & jnp.uint32(1))
    return r & jnp.uint32(0xFFFF0000)


def _mm_body(tlo_ref, thi_ref, w1_ref, w2_ref, o_ref):
    dn = (((0,), (0,)), ((), ()))
    f32 = jnp.float32
    d1_lo = jax.lax.dot_general(tlo_ref[...], w1_ref[...], dn,
                                preferred_element_type=f32)
    d2_lo = jax.lax.dot_general(tlo_ref[...], w2_ref[...], dn,
                                preferred_element_type=f32)
    d1_hi = jax.lax.dot_general(thi_ref[...], w1_ref[...], dn,
                                preferred_element_type=f32)
    d2_hi = jax.lax.dot_general(thi_ref[...], w2_ref[...], dn,
                                preferred_element_type=f32)
    cont_lo = (_rne16(d1_lo) >> jnp.uint32(16)) | _rne16(d2_lo)
    cont_hi = (_rne16(d1_hi) >> jnp.uint32(16)) | _rne16(d2_hi)
    o_ref[...] = jnp.concatenate([cont_lo, cont_hi], axis=1)


def _score_table(table_t, w1, w2):
    nb = _H // _BM
    return pl.pallas_call(
        _mm_body,
        grid=(nb,),
        in_specs=[
            pl.BlockSpec((_D, _BM), lambda i: (0, i)),
            pl.BlockSpec((_D, _BM),
                         lambda i: (0, jnp.minimum(i + _H // _BM,
                                                   (_VOCAB - 1) // _BM))),
            pl.BlockSpec((_D, 64), lambda i: (0, 0)),
            pl.BlockSpec((_D, 64), lambda i: (0, 0)),
        ],
        out_specs=pl.BlockSpec((_BM, _CP), lambda i: (i, 0)),
        out_shape=jax.ShapeDtypeStruct((_H, _CP), jnp.uint32),
    )(table_t, table_t, w1, w2)


def _gather_reduce(q_flat, idx, bvec):
    mesh = plsc.VectorSubcoreMesh(core_axis_name="c", subcore_axis_name="s")

    @functools.partial(
        pl.kernel,
        mesh=mesh,
        out_type=jax.ShapeDtypeStruct((_B,), jnp.float32),
        scratch_types=[
            pltpu.VMEM((_NG, _G), jnp.int32),
            pltpu.VMEM((_NG * _G,), jnp.uint32),
            pltpu.VMEM((16,), jnp.float32),
            pltpu.VMEM((_BPW,), jnp.float32),
            pltpu.SemaphoreType.DMA,
        ],
    )
    def k(q_hbm, idx_hbm, b_hbm, out_hbm, idx_v, g_v, b_v, out_v, sem):
        wid = lax.axis_index("s") * _NC + lax.axis_index("c")
        pltpu.sync_copy(idx_hbm.at[wid], idx_v)
        pltpu.sync_copy(b_hbm, b_v)

        def fire_group(gi, carry):
            cps = []
            for j in range(_K):
                row = gi * _K + j
                cps.append(pltpu.async_copy(
                    q_hbm.at[idx_v.at[row]],
                    g_v.at[pl.ds(row * _G, _G)],
                    sem,
                ))
            for cp in cps:
                cp.wait()
            return carry

        lax.fori_loop(0, _NG // _K, fire_group, 0)

        def col(c, carry):
            def red_lo(l, acc):
                g = g_v[pl.ds(l * _BPW + c * 16, 16)]
                return acc + jax.lax.bitcast_convert_type(
                    g << jnp.uint32(16), jnp.float32)

            def red_hi(l, acc):
                g = g_v[pl.ds(l * _BPW + c * 16, 16)]
                return acc + jax.lax.bitcast_convert_type(
                    g & jnp.uint32(0xFFFF0000), jnp.float32)

            acc = lax.fori_loop(0, _L, red_lo, b_v[...])
            acc = lax.fori_loop(_L, _C, red_hi, acc)
            out_v[pl.ds(c * 16, 16)] = 1.0 / (1.0 + jnp.exp(-acc))
            return carry

        lax.fori_loop(0, _BPW // 16, col, 0)
        pltpu.sync_copy(out_v, out_hbm.at[pl.ds(wid * _BPW, _BPW)])

    return k(q_flat, idx, bvec)


def kernel(input1, input2, table, W, b):
    wt = W.reshape(_C, _D).T.astype(jnp.float32)
    w1 = jnp.pad(wt[:, :_L], ((0, 0), (0, 64 - _L)))
    w2 = jnp.pad(wt[:, _L:], ((0, 0), (0, 64 - _L)))
    table_t = table.astype(jnp.float32).T
    Q = _score_table(table_t, w1, w2)

    pos = jnp.arange(_L, dtype=jnp.int32)
    base1 = jnp.where(input1 < _H, input1 * _CP, (input1 - _H) * _CP + 64)
    base2 = jnp.where(input2 < _H, input2 * _CP, (input2 - _H) * _CP + 64)
    idx1 = base1 + pos[None, :]
    idx2 = base2 + pos[None, :]
    idx_all = jnp.concatenate([idx1, idx2], axis=1)
    idx_r = (idx_all.reshape(_NW, _BPW, _C)
             .transpose(0, 2, 1)
             .reshape(_NW, _NG, _G))

    bvec = jnp.broadcast_to(b.astype(jnp.float32), (16,))
    out = _gather_reduce(Q.reshape(-1), idx_r, bvec)
    return out.reshape(_B, 1)


# speedup vs baseline: 1.0568x; 1.0568x over previous
import functools

import jax
import jax.numpy as jnp
from jax import lax
from jax.experimental import pallas as pl
from jax.experimental.pallas import tpu as pltpu
from jax.experimental.pallas import tpu_sc as plsc

_VOCAB = 1000001
_H = 501760
_B = 16384
_L = 50
_D = 300
_C = 2 * _L
_CP = 128
_BM = 2048

_NC = 2
_NS = 16
_NW = _NC * _NS
_BPW = _B // _NW
_G = 128
_NG = _C * _BPW // _G
_K = 16


def _rne16(d):
    u = jax.lax.bitcast_convert_type(d, jnp.uint32)
    r = u + jnp.uint32(0x7FFF) + ((u >> jnp.uint32(16)) & jnp.uint32(1))
    return r & jnp.uint32(0xFFFF0000)


def _pack_half(d):
    d1 = jax.lax.slice(d, (0, 0), (_BM, 64))
    d2 = jax.lax.slice(d, (0, 64), (_BM, 128))
    return (_rne16(d1) >> jnp.uint32(16)) | _rne16(d2)


def _mm_body(tlo_ref, thi_ref, w_ref, o_ref):
    dn = (((0,), (0,)), ((), ()))
    f32 = jnp.float32
    d_lo = jax.lax.dot_general(tlo_ref[...], w_ref[...], dn,
                               preferred_element_type=f32)
    d_hi = jax.lax.dot_general(thi_ref[...], w_ref[...], dn,
                               preferred_element_type=f32)
    o_ref[...] = jnp.concatenate([_pack_half(d_lo), _pack_half(d_hi)],
                                 axis=1)


def _score_table(table_t, wcat):
    nb = _H // _BM
    return pl.pallas_call(
        _mm_body,
        grid=(nb,),
        in_specs=[
            pl.BlockSpec((_D, _BM), lambda i: (0, i)),
            pl.BlockSpec((_D, _BM),
                         lambda i: (0, jnp.minimum(i + _H // _BM,
                                                   (_VOCAB - 1) // _BM))),
            pl.BlockSpec((_D, _CP), lambda i: (0, 0)),
        ],
        out_specs=pl.BlockSpec((_BM, _CP), lambda i: (i, 0)),
        out_shape=jax.ShapeDtypeStruct((_H, _CP), jnp.uint32),
    )(table_t, table_t, wcat)


def _gather_reduce(q_flat, idx, bvec):
    mesh = plsc.VectorSubcoreMesh(core_axis_name="c", subcore_axis_name="s")

    @functools.partial(
        pl.kernel,
        mesh=mesh,
        out_type=jax.ShapeDtypeStruct((_B,), jnp.float32),
        scratch_types=[
            pltpu.VMEM((_NG, _G), jnp.int32),
            pltpu.VMEM((_NG * _G,), jnp.uint32),
            pltpu.VMEM((16,), jnp.float32),
            pltpu.VMEM((_BPW,), jnp.float32),
            pltpu.SemaphoreType.DMA,
        ],
    )
    def k(q_hbm, idx_hbm, b_hbm, out_hbm, idx_v, g_v, b_v, out_v, sem):
        wid = lax.axis_index("s") * _NC + lax.axis_index("c")
        pltpu.sync_copy(idx_hbm.at[wid], idx_v)
        pltpu.sync_copy(b_hbm, b_v)

        def fire_group(gi, carry):
            cps = []
            for j in range(_K):
                row = gi * _K + j
                cps.append(pltpu.async_copy(
                    q_hbm.at[idx_v.at[row]],
                    g_v.at[pl.ds(row * _G, _G)],
                    sem,
                ))
            for cp in cps:
                cp.wait()
            return carry

        lax.fori_loop(0, _NG // _K, fire_group, 0)

        def col(c, carry):
            def red_lo(l, acc):
                g = g_v[pl.ds(l * _BPW + c * 16, 16)]
                return acc + jax.lax.bitcast_convert_type(
                    g << jnp.uint32(16), jnp.float32)

            def red_hi(l, acc):
                g = g_v[pl.ds(l * _BPW + c * 16, 16)]
                return acc + jax.lax.bitcast_convert_type(
                    g & jnp.uint32(0xFFFF0000), jnp.float32)

            acc = lax.fori_loop(0, _L, red_lo, b_v[...])
            acc = lax.fori_loop(_L, _C, red_hi, acc)
            out_v[pl.ds(c * 16, 16)] = 1.0 / (1.0 + jnp.exp(-acc))
            return carry

        lax.fori_loop(0, _BPW // 16, col, 0)
        pltpu.sync_copy(out_v, out_hbm.at[pl.ds(wid * _BPW, _BPW)])

    return k(q_flat, idx, bvec)


def kernel(input1, input2, table, W, b):
    wt = W.reshape(_C, _D).T.astype(jnp.float32)
    w1 = jnp.pad(wt[:, :_L], ((0, 0), (0, 64 - _L)))
    w2 = jnp.pad(wt[:, _L:], ((0, 0), (0, 64 - _L)))
    wcat = jnp.concatenate([w1, w2], axis=1)
    table_t = table.astype(jnp.float32).T
    Q = _score_table(table_t, wcat)

    pos = jnp.arange(_L, dtype=jnp.int32)
    base1 = jnp.where(input1 < _H, input1 * _CP, (input1 - _H) * _CP + 64)
    base2 = jnp.where(input2 < _H, input2 * _CP, (input2 - _H) * _CP + 64)
    idx1 = base1 + pos[None, :]
    idx2 = base2 + pos[None, :]
    idx_all = jnp.concatenate([idx1, idx2], axis=1)
    idx_r = (idx_all.reshape(_NW, _BPW, _C)
             .transpose(0, 2, 1)
             .reshape(_NW, _NG, _G))

    bvec = jnp.broadcast_to(b.astype(jnp.float32), (16,))
    out = _gather_reduce(Q.reshape(-1), idx_r, bvec)
    return out.reshape(_B, 1)
